# baseline (speedup 1.0000x reference)
import jax
import jax.numpy as jnp
from jax.experimental import pallas as pl
from jax.experimental.pallas import tpu as pltpu


def kernel(input, index, value):
    M, D1, D2 = input.shape
    B = index.shape[0]

    def body(idx_ref, in_hbm, val_blk, out_blk):
        del idx_ref, in_hbm
        out_blk[...] = val_blk[...]

    grid_spec = pltpu.PrefetchScalarGridSpec(
        num_scalar_prefetch=1,
        grid=(B,),
        in_specs=[
            pl.BlockSpec(memory_space=pl.ANY),
            pl.BlockSpec((1, D1, D2), lambda b, idx: (b, 0, 0)),
        ],
        out_specs=pl.BlockSpec((1, D1, D2), lambda b, idx: (idx[b], 0, 0)),
    )
    return pl.pallas_call(
        body,
        grid_spec=grid_spec,
        out_shape=jax.ShapeDtypeStruct((M, D1, D2), input.dtype),
        input_output_aliases={1: 0},
    )(index, input, value)


# speedup vs baseline: 1.7023x; 1.7023x over previous
import functools

import jax
import jax.numpy as jnp
from jax import lax
from jax.experimental import pallas as pl
from jax.experimental.pallas import tpu as pltpu
from jax.experimental.pallas import tpu_sc as plsc

_NC = 2
_NS = 16
_NW = _NC * _NS
_L = 16
_K = 32


def _sc_scatter_body(M, B, rpw, rpw_pad, idx_hbm, val_hbm, out_ref,
                     idx_v, lastpos_v, selb_v, seldst_v, sem_s):
    wid = lax.axis_index("s") * _NC + lax.axis_index("c")
    base = wid * rpw

    pltpu.sync_copy(idx_hbm, idx_v)

    minus1 = jnp.full((_L,), -1, jnp.int32)

    def init_body(i, _):
        lastpos_v[pl.ds(i * _L, _L)] = minus1
        return 0

    lax.fori_loop(0, rpw_pad // _L, init_body, 0, unroll=4)

    iota = lax.iota(jnp.int32, _L)

    def scan_body(i, _):
        v = idx_v[pl.ds(i * _L, _L)]
        owned = (v >= base) & (v < base + rpw)
        _, lastmask = plsc.scan_count(v, owned)
        keep = lastmask & owned
        bvec = iota + i * _L
        plsc.store_scatter(lastpos_v, [v - base], bvec, mask=keep)
        return 0

    lax.fori_loop(0, B // _L, scan_body, 0, unroll=4)

    def compact_body(i, off):
        lp = lastpos_v[pl.ds(i * _L, _L)]
        m = lp >= 0
        plsc.store_compressed(selb_v.at[pl.ds(off, _L)], lp, mask=m)
        plsc.store_compressed(
            seldst_v.at[pl.ds(off, _L)], iota + (base + i * _L), mask=m)
        return off + jnp.sum(m.astype(jnp.int32))

    cnt = lax.fori_loop(0, rpw_pad // _L, compact_body, 0, unroll=4)

    @pl.when(cnt > 0)
    def _move():
        def chunk_body(c, _):
            o = c * _L
            bv = selb_v[pl.ds(o, _L)]
            dv = seldst_v[pl.ds(o, _L)]
            for j in range(_L):
                @pl.when(o + j < cnt)
                def _fire():
                    pltpu.async_copy(
                        val_hbm.at[bv[j]], out_ref.at[dv[j]], sem_s)
            for j in range(_L):
                @pl.when(o + j < cnt)
                def _drain():
                    pltpu.make_async_copy(
                        val_hbm.at[bv[j]], out_ref.at[dv[j]], sem_s).wait()
            return 0

        lax.fori_loop(0, (cnt + _L - 1) // _L, chunk_body, 0)


def kernel(input, index, value):
    M, D1, D2 = input.shape
    B = index.shape[0]
    rpw = (M + _NW - 1) // _NW
    rpw_pad = ((rpw + _L - 1) // _L) * _L
    cap = rpw_pad + _K

    mesh = plsc.VectorSubcoreMesh(core_axis_name="c", subcore_axis_name="s")
    sc_call = pl.kernel(
        functools.partial(_sc_scatter_body, M, B, rpw, rpw_pad),
        out_type=(),
        mesh=mesh,
        compiler_params=pltpu.CompilerParams(needs_layout_passes=False),
        scratch_types=[
            pltpu.VMEM((B,), jnp.int32),
            pltpu.VMEM((rpw_pad,), jnp.int32),
            pltpu.VMEM((cap,), jnp.int32),
            pltpu.VMEM((cap,), jnp.int32),
            pltpu.SemaphoreType.DMA,
        ],
    )

    out_ref = jax.new_ref(input)
    sc_call(index, value, out_ref)
    return out_ref[...]


# speedup vs baseline: 6.0179x; 3.5351x over previous
import functools

import jax
import jax.numpy as jnp
from jax import lax
from jax.experimental import pallas as pl
from jax.experimental.pallas import tpu as pltpu
from jax.experimental.pallas import tpu_sc as plsc

_NC = 2
_NS = 16
_NW = _NC * _NS
_L = 16
_K = 32


def _sc_scatter_body(M, B, rpw, rpw_pad, idx_hbm, val_hbm, out_ref,
                     idx_v, lastpos_v, selb_v, seldst_v, buf_v, sem_g, sem_s):
    wid = lax.axis_index("s") * _NC + lax.axis_index("c")
    base = wid * rpw

    pltpu.sync_copy(idx_hbm, idx_v)

    minus1 = jnp.full((_L,), -1, jnp.int32)

    def init_body(i, _):
        lastpos_v[pl.ds(i * _L, _L)] = minus1
        return 0

    lax.fori_loop(0, rpw_pad // _L, init_body, 0, unroll=4)

    iota = lax.iota(jnp.int32, _L)

    def scan_body(i, _):
        v = idx_v[pl.ds(i * _L, _L)]
        owned = (v >= base) & (v < base + rpw)
        _, lastmask = plsc.scan_count(v, owned)
        keep = lastmask & owned
        bvec = iota + i * _L
        plsc.store_scatter(lastpos_v, [v - base], bvec, mask=keep)
        return 0

    lax.fori_loop(0, B // _L, scan_body, 0, unroll=4)

    def compact_body(i, off):
        lp = lastpos_v[pl.ds(i * _L, _L)]
        m = lp >= 0
        plsc.store_compressed(selb_v.at[pl.ds(off, _L)], lp, mask=m)
        plsc.store_compressed(
            seldst_v.at[pl.ds(off, _L)], iota + (base + i * _L), mask=m)
        return off + jnp.sum(m.astype(jnp.int32))

    cnt = lax.fori_loop(0, rpw_pad // _L, compact_body, 0, unroll=4)

    @pl.when(cnt > 0)
    def _move():
        def chunk_body(c, _):
            o = c * _L
            bv = selb_v[pl.ds(o, _L)]
            dv = seldst_v[pl.ds(o, _L)]
            slot = c % 2
            for j in range(_L):
                @pl.when(o + j < cnt)
                def _fire_gather():
                    pltpu.async_copy(
                        val_hbm.at[bv[j]], buf_v.at[slot, j], sem_g)

            @pl.when(c > 0)
            def _drain_prev_scatters():
                po = (c - 1) * _L
                pbv = selb_v[pl.ds(po, _L)]
                pdv = seldst_v[pl.ds(po, _L)]
                for j in range(_L):
                    @pl.when(po + j < cnt)
                    def _drain_s():
                        pltpu.make_async_copy(
                            buf_v.at[1 - slot, j], out_ref.at[pdv[j]],
                            sem_s).wait()

            for j in range(_L):
                @pl.when(o + j < cnt)
                def _drain_g():
                    pltpu.make_async_copy(
                        val_hbm.at[bv[j]], buf_v.at[slot, j], sem_g).wait()
            for j in range(_L):
                @pl.when(o + j < cnt)
                def _fire_scatter():
                    pltpu.async_copy(
                        buf_v.at[slot, j], out_ref.at[dv[j]], sem_s)
            return 0

        nchunk = (cnt + _L - 1) // _L
        lax.fori_loop(0, nchunk, chunk_body, 0)

        fo = (nchunk - 1) * _L
        fbv = selb_v[pl.ds(fo, _L)]
        fdv = seldst_v[pl.ds(fo, _L)]
        fslot = (nchunk - 1) % 2
        for j in range(_L):
            @pl.when(fo + j < cnt)
            def _drain_final():
                pltpu.make_async_copy(
                    buf_v.at[fslot, j], out_ref.at[fdv[j]], sem_s).wait()


def kernel(input, index, value):
    M, D1, D2 = input.shape
    B = index.shape[0]
    rpw = (M + _NW - 1) // _NW
    rpw_pad = ((rpw + _L - 1) // _L) * _L
    cap = rpw_pad + _K

    mesh = plsc.VectorSubcoreMesh(core_axis_name="c", subcore_axis_name="s")
    sc_call = pl.kernel(
        functools.partial(_sc_scatter_body, M, B, rpw, rpw_pad),
        out_type=(),
        mesh=mesh,
        compiler_params=pltpu.CompilerParams(needs_layout_passes=False),
        scratch_types=[
            pltpu.VMEM((B,), jnp.int32),
            pltpu.VMEM((rpw_pad,), jnp.int32),
            pltpu.VMEM((cap,), jnp.int32),
            pltpu.VMEM((cap,), jnp.int32),
            pltpu.VMEM((2, _L, D1, D2), jnp.float32),
            pltpu.SemaphoreType.DMA,
            pltpu.SemaphoreType.DMA,
        ],
    )

    out_ref = jax.new_ref(input)
    sc_call(index, value, out_ref)
    return out_ref[...]
